# Initial kernel scaffold; baseline (speedup 1.0000x reference)
import jax
import jax.numpy as jnp
from jax.experimental import pallas as pl


def kernel(inputs, edge_index, edge_weight, W, b):
    raise NotImplementedError("write your pallas kernel here")


# speedup vs baseline: 4.7131x; 4.7131x over previous
import functools

import jax
import jax.numpy as jnp
from jax import lax
from jax.experimental import pallas as pl
from jax.experimental.pallas import tpu as pltpu
from jax.experimental.pallas import tpu_sc as plsc

N = 10000
E = 320000
C = 128
K = 128
NC = 2
NS = 16
NW = NC * NS
NBLK = E // K
ROWS_PER_TILE = 624
ZR = 104


def _spmm_body(scale, x_hbm, src_hbm, dst_hbm, w_hbm, part_hbm,
               acc, src_v, dst_v, w_v, rows_v, zero_v, sem):
    cid = lax.axis_index("c")
    sid = lax.axis_index("s")
    wid = sid * NC + cid

    def zbody(r, _):
        for j in range(C // 16):
            zero_v[r, pl.ds(16 * j, 16)] = jnp.zeros((16,), jnp.float32)
        return 0
    lax.fori_loop(0, ZR, zbody, 0)
    for q in range(ROWS_PER_TILE // ZR):
        pltpu.sync_copy(zero_v, acc.at[pl.ds(sid * ROWS_PER_TILE + q * ZR, ZR)])

    @pl.when(sid == NS - 1)
    def _():
        pltpu.sync_copy(zero_v.at[pl.ds(0, 16)],
                        acc.at[pl.ds(NS * ROWS_PER_TILE, N - NS * ROWS_PER_TILE)])
    plsc.subcore_barrier()

    nblk = (NBLK - wid + NW - 1) // NW

    def block(i, _):
        base = (wid + i * NW) * K
        pltpu.sync_copy(src_hbm.at[pl.ds(base, K)], src_v)
        pltpu.sync_copy(dst_hbm.at[pl.ds(base, K)], dst_v)
        pltpu.sync_copy(w_hbm.at[pl.ds(base, K)], w_v)
        pltpu.async_copy(x_hbm.at[src_v], rows_v, sem).wait()

        def mbody(t, _):
            w16 = w_v[pl.ds(16 * t, 16)] * scale
            for l in range(16):
                ws = w16[l]
                e = 16 * t + l
                for j in range(C // 16):
                    rows_v[e, pl.ds(16 * j, 16)] = rows_v[e, pl.ds(16 * j, 16)] * ws
            return 0
        lax.fori_loop(0, K // 16, mbody, 0)

        pltpu.sync_copy(rows_v, acc.at[dst_v], add=True)
        return 0
    lax.fori_loop(0, nblk, block, 0)

    plsc.subcore_barrier()
    pltpu.sync_copy(acc.at[pl.ds(sid * ROWS_PER_TILE, ROWS_PER_TILE)],
                    part_hbm.at[cid, pl.ds(sid * ROWS_PER_TILE, ROWS_PER_TILE)])

    @pl.when(sid == NS - 1)
    def _():
        tail = NS * ROWS_PER_TILE
        pltpu.sync_copy(acc.at[pl.ds(tail, N - tail)],
                        part_hbm.at[cid, pl.ds(tail, N - tail)])


@functools.lru_cache(maxsize=None)
def _make_spmm(scale):
    mesh = plsc.VectorSubcoreMesh(core_axis_name="c", subcore_axis_name="s")
    return pl.kernel(
        functools.partial(_spmm_body, scale),
        out_type=jax.ShapeDtypeStruct((NC, N, C), jnp.float32),
        mesh=mesh,
        scratch_types=[
            pltpu.VMEM_SHARED((N, C), jnp.float32),
            pltpu.VMEM((K,), jnp.int32),
            pltpu.VMEM((K,), jnp.int32),
            pltpu.VMEM((K,), jnp.float32),
            pltpu.VMEM((K, C), jnp.float32),
            pltpu.VMEM((ZR, C), jnp.float32),
            pltpu.SemaphoreType.DMA,
        ],
    )

_ROWS_BLK = 1000
_GRID = N // _ROWS_BLK


def _combine_body(p0_ref, p1_ref, prev_ref, o_ref):
    o_ref[...] = p0_ref[...] + p1_ref[...] - prev_ref[...]


_combine = pl.pallas_call(
    _combine_body,
    grid=(_GRID,),
    in_specs=[pl.BlockSpec((_ROWS_BLK, C), lambda i: (i, 0))] * 3,
    out_specs=pl.BlockSpec((_ROWS_BLK, C), lambda i: (i, 0)),
    out_shape=jax.ShapeDtypeStruct((N, C), jnp.float32),
)


def _combine2_body(p0_ref, p1_ref, o_ref):
    o_ref[...] = p0_ref[...] + p1_ref[...]


_combine2 = pl.pallas_call(
    _combine2_body,
    grid=(_GRID,),
    in_specs=[pl.BlockSpec((_ROWS_BLK, C), lambda i: (i, 0))] * 2,
    out_specs=pl.BlockSpec((_ROWS_BLK, C), lambda i: (i, 0)),
    out_shape=jax.ShapeDtypeStruct((N, C), jnp.float32),
)


def _final_body(c0_ref, c1_ref, c2_ref, c3_ref, w_ref, b_ref, o_ref):
    acc = jnp.dot(c0_ref[...], w_ref[0], preferred_element_type=jnp.float32)
    acc += jnp.dot(c1_ref[...], w_ref[1], preferred_element_type=jnp.float32)
    acc += jnp.dot(c2_ref[...], w_ref[2], preferred_element_type=jnp.float32)
    acc += jnp.dot(c3_ref[...], w_ref[3], preferred_element_type=jnp.float32)
    o_ref[...] = jax.nn.relu(acc + b_ref[...])


_final = pl.pallas_call(
    _final_body,
    grid=(_GRID,),
    in_specs=[pl.BlockSpec((_ROWS_BLK, C), lambda i: (i, 0))] * 4
    + [pl.BlockSpec((4, C, C), lambda i: (0, 0, 0)),
       pl.BlockSpec((1, C), lambda i: (0, 0))],
    out_specs=pl.BlockSpec((_ROWS_BLK, C), lambda i: (i, 0)),
    out_shape=jax.ShapeDtypeStruct((N, C), jnp.float32),
)


def kernel(inputs, edge_index, edge_weight, W, b):
    x = inputs[0]
    dst = edge_index[0]
    src = edge_index[1]

    _spmm_1 = _make_spmm(1.0)
    _spmm_2 = _make_spmm(2.0)

    p1 = _spmm_1(x, src, dst, edge_weight)
    c1 = _combine2(p1[0], p1[1])
    p2 = _spmm_2(c1, src, dst, edge_weight)
    c2 = _combine(p2[0], p2[1], x)
    p3 = _spmm_2(c2, src, dst, edge_weight)
    c3 = _combine(p3[0], p3[1], c1)

    out = _final(x, c1, c2, c3, W, b.reshape(1, C))
    return out[None]
